# baseline (speedup 1.0000x reference)
import functools

import jax
import jax.numpy as jnp
from jax import lax
from jax.experimental import pallas as pl
from jax.experimental.pallas import tpu as pltpu
from jax.experimental.pallas import tpu_sc as plsc

_NC = 2
_NS = 16
_NW = _NC * _NS
_RPD = 128
_G = 10
_EPS = 1e-5


def _sc_gather(idx3, tab2, n_rows):
    dma_per_w = idx3.shape[1]
    chunks = dma_per_w // _G
    rows_per_chunk = _G * _RPD
    rows_per_w = dma_per_w * _RPD
    D = tab2.shape[1]

    mesh = plsc.VectorSubcoreMesh(core_axis_name="c", subcore_axis_name="s")

    @functools.partial(
        pl.kernel,
        out_type=jax.ShapeDtypeStruct((n_rows, D), jnp.float32),
        mesh=mesh,
        compiler_params=pltpu.CompilerParams(use_tc_tiling_on_sc=False),
        scratch_types=[
            pltpu.VMEM((dma_per_w, _RPD), jnp.int32),
            pltpu.VMEM((rows_per_chunk, D), jnp.float32),
            pltpu.SemaphoreType.DMA,
        ],
    )
    def k(idx_hbm, tab_hbm, out_hbm, idx_v, rows_v, sem):
        wid = lax.axis_index("s") * _NC + lax.axis_index("c")
        pltpu.sync_copy(idx_hbm.at[wid], idx_v)

        def chunk(c, carry):
            handles = []
            for j in range(_G):
                h = pltpu.async_copy(
                    tab_hbm.at[idx_v.at[c * _G + j]],
                    rows_v.at[pl.ds(j * _RPD, _RPD)],
                    sem,
                )
                handles.append(h)
            for h in handles:
                h.wait()
            base = wid * rows_per_w + c * rows_per_chunk
            pltpu.sync_copy(rows_v, out_hbm.at[pl.ds(base, rows_per_chunk)])
            return carry

        lax.fori_loop(0, chunks, chunk, 0)

    return k(idx3, tab2)


def _tc_ln_concat(cat3, xn2, num_w, num_b, gc, bc, gn, bn):
    T, Fc, D = cat3.shape
    Fn = xn2.shape[1]
    TB = 256
    grid = (T // TB,)

    def body(cat_ref, xn_ref, nw_ref, nb_ref, gc_ref, bc_ref, gn_ref, bn_ref,
             out_ref):
        xc = cat_ref[...]
        mu = jnp.mean(xc, axis=-1, keepdims=True)
        d = xc - mu
        var = jnp.mean(d * d, axis=-1, keepdims=True)
        yc = d * lax.rsqrt(var + _EPS) * gc_ref[...][None] + bc_ref[...][None]

        xn = xn_ref[...][:, :, None]
        e = xn * nw_ref[...][None] + nb_ref[...][None]
        mun = jnp.mean(e, axis=-1, keepdims=True)
        dn = e - mun
        varn = jnp.mean(dn * dn, axis=-1, keepdims=True)
        yn = dn * lax.rsqrt(varn + _EPS) * gn_ref[...][None] + bn_ref[...][None]

        out_ref[...] = jnp.concatenate([yc, yn], axis=1)

    return pl.pallas_call(
        body,
        grid=grid,
        in_specs=[
            pl.BlockSpec((TB, Fc, D), lambda i: (i, 0, 0)),
            pl.BlockSpec((TB, Fn), lambda i: (i, 0)),
            pl.BlockSpec((Fn, D), lambda i: (0, 0)),
            pl.BlockSpec((Fn, D), lambda i: (0, 0)),
            pl.BlockSpec((1, D), lambda i: (0, 0)),
            pl.BlockSpec((1, D), lambda i: (0, 0)),
            pl.BlockSpec((1, D), lambda i: (0, 0)),
            pl.BlockSpec((1, D), lambda i: (0, 0)),
        ],
        out_specs=pl.BlockSpec((TB, Fc + Fn, D), lambda i: (i, 0, 0)),
        out_shape=jax.ShapeDtypeStruct((T, Fc + Fn, D), jnp.float32),
    )(cat3, xn2, num_w, num_b, gc, bc, gn, bn)


def kernel(x_cat, x_num, tables, num_w, num_b, cat_ln_g, cat_ln_b, num_ln_g,
           num_ln_b):
    B, O, Fc = x_cat.shape
    Fn = x_num.shape[-1]
    V, D = tables.shape[1], tables.shape[2]
    T = B * O
    n_rows = T * Fc

    idx = x_cat.reshape(T, Fc).astype(jnp.int32) + jnp.arange(
        Fc, dtype=jnp.int32) * jnp.int32(V)
    idx3 = idx.reshape(_NW, n_rows // (_NW * _RPD), _RPD)
    tab2 = tables.reshape(Fc * V, D)

    gathered = _sc_gather(idx3, tab2, n_rows)
    out = _tc_ln_concat(
        gathered.reshape(T, Fc, D),
        x_num.reshape(T, Fn),
        num_w, num_b,
        cat_ln_g.reshape(1, D), cat_ln_b.reshape(1, D),
        num_ln_g.reshape(1, D), num_ln_b.reshape(1, D),
    )
    return out.reshape(B, O, Fc + Fn, D)
